# Initial kernel scaffold; baseline (speedup 1.0000x reference)
import jax
import jax.numpy as jnp
from jax.experimental import pallas as pl


def kernel(x, edge_index, W1, att_src1, att_dst1, b1, W2, att_src2, att_dst2, b2, Wl, bl):
    raise NotImplementedError("write your pallas kernel here")


# speedup vs baseline: 3.4539x; 3.4539x over previous
import jax
import jax.numpy as jnp
from jax import lax
from jax.experimental import pallas as pl
from jax.experimental.pallas import tpu as pltpu
from jax.experimental.pallas import tpu_sc as plsc

N_NODES = 10000
N_EDGES = 320000
D_IN = 128
HEADS = 4
CH = 16
HID = HEADS * CH
N_CLASSES = 2

N_PAD = 10112
GW = 80
AW = 16
ACCW = 80

NC = 2
NS = 16
HALF = N_PAD // NC
ROWS_PER_TILE = HALF // NS
E_PER_SUB = N_EDGES // NS
CHUNK = 400
N_CHUNKS = E_PER_SUB // CHUNK

BLK = N_PAD // 16
N_BLKS = 16


def _tc_prep_kernel(x_ref, w_ref, p_ref, pd_ref, g_ref, ad_ref):
    h = jnp.dot(x_ref[...], w_ref[...], preferred_element_type=jnp.float32)
    g_ref[...] = jnp.dot(h, p_ref[...], preferred_element_type=jnp.float32)
    ad_ref[...] = jnp.dot(h, pd_ref[...], preferred_element_type=jnp.float32)


def _tc_prep(x_pad, w, p, pd, d_in):
    return pl.pallas_call(
        _tc_prep_kernel,
        grid=(N_BLKS,),
        in_specs=[
            pl.BlockSpec((BLK, d_in), lambda i: (i, 0)),
            pl.BlockSpec((d_in, HID), lambda i: (0, 0)),
            pl.BlockSpec((HID, GW), lambda i: (0, 0)),
            pl.BlockSpec((HID, AW), lambda i: (0, 0)),
        ],
        out_specs=[
            pl.BlockSpec((BLK, GW), lambda i: (i, 0)),
            pl.BlockSpec((BLK, AW), lambda i: (i, 0)),
        ],
        out_shape=[
            jax.ShapeDtypeStruct((N_PAD, GW), jnp.float32),
            jax.ShapeDtypeStruct((N_PAD, AW), jnp.float32),
        ],
    )(x_pad, w, p, pd)


def _tc_mid_kernel(acc_ref, e16_ref, b_ref, w_ref, p_ref, pd_ref,
                   g_ref, ad_ref):
    a = acc_ref[...]
    o = a[:, :HID]
    d = a[:, 76:80]
    r = 1.0 / (d + 1e-16)
    rex = jnp.dot(r, e16_ref[...], preferred_element_type=jnp.float32)
    h1 = jnp.maximum(o * rex + b_ref[...], 0.0)
    t = jnp.dot(h1, w_ref[...], preferred_element_type=jnp.float32)
    g_ref[...] = jnp.dot(t, p_ref[...], preferred_element_type=jnp.float32)
    ad_ref[...] = jnp.dot(t, pd_ref[...], preferred_element_type=jnp.float32)


def _tc_mid(acc, e16, b, w, p, pd):
    return pl.pallas_call(
        _tc_mid_kernel,
        grid=(N_BLKS,),
        in_specs=[
            pl.BlockSpec((BLK, ACCW), lambda i: (i, 0)),
            pl.BlockSpec((HEADS, HID), lambda i: (0, 0)),
            pl.BlockSpec((1, HID), lambda i: (0, 0)),
            pl.BlockSpec((HID, HID), lambda i: (0, 0)),
            pl.BlockSpec((HID, GW), lambda i: (0, 0)),
            pl.BlockSpec((HID, AW), lambda i: (0, 0)),
        ],
        out_specs=[
            pl.BlockSpec((BLK, GW), lambda i: (i, 0)),
            pl.BlockSpec((BLK, AW), lambda i: (i, 0)),
        ],
        out_shape=[
            jax.ShapeDtypeStruct((N_PAD, GW), jnp.float32),
            jax.ShapeDtypeStruct((N_PAD, AW), jnp.float32),
        ],
    )(acc, e16, b, w, p, pd)


def _tc_final_kernel(acc_ref, e16_ref, b_ref, wl_ref, bl_ref, out_ref):
    a = acc_ref[...]
    o = a[:, :HID]
    d = a[:, 76:80]
    r = 1.0 / (d + 1e-16)
    rex = jnp.dot(r, e16_ref[...], preferred_element_type=jnp.float32)
    z = o * rex + b_ref[...]
    h2 = jnp.maximum(z, 0.01 * z)
    out_ref[...] = (
        jnp.dot(h2, wl_ref[...], preferred_element_type=jnp.float32)
        + bl_ref[...]
    )


def _tc_final(acc, e16, b, wl_pad, bl_pad):
    return pl.pallas_call(
        _tc_final_kernel,
        grid=(N_BLKS,),
        in_specs=[
            pl.BlockSpec((BLK, ACCW), lambda i: (i, 0)),
            pl.BlockSpec((HEADS, HID), lambda i: (0, 0)),
            pl.BlockSpec((1, HID), lambda i: (0, 0)),
            pl.BlockSpec((HID, 128), lambda i: (0, 0)),
            pl.BlockSpec((1, 128), lambda i: (0, 0)),
        ],
        out_specs=pl.BlockSpec((BLK, 128), lambda i: (i, 0)),
        out_shape=jax.ShapeDtypeStruct((N_PAD, 128), jnp.float32),
    )(acc, e16, b, wl_pad, bl_pad)


def _sc_edge_body(g_hbm, ad_hbm, src_hbm, dst_hbm, acc_hbm,
                  g_v, ad_v, msg_v, si_v, di_v, sim_v, dil_v, mf_v, stage_v,
                  acc_sp):
    cid = lax.axis_index("c")
    sid = lax.axis_index("s")

    zero16 = jnp.zeros((16,), jnp.float32)

    def zrow(rr, _):
        for k in range(ACCW // 16):
            stage_v[rr, pl.ds(k * 16, 16)] = zero16
        return 0
    lax.fori_loop(0, ROWS_PER_TILE, zrow, 0, unroll=4)

    def zg(rr, _):
        for k in range(GW // 16):
            g_v[rr, pl.ds(k * 16, 16)] = zero16
        return 0
    lax.fori_loop(0, CHUNK, zg, 0, unroll=4)

    pltpu.sync_copy(stage_v,
                    acc_sp.at[pl.ds(sid * ROWS_PER_TILE, ROWS_PER_TILE), :])
    plsc.subcore_barrier()

    wbase = sid * E_PER_SUB
    base = cid * HALF

    def chunk_body(ch, _):
        eb = wbase + ch * CHUNK
        pltpu.sync_copy(src_hbm.at[pl.ds(eb, CHUNK)], si_v)
        pltpu.sync_copy(dst_hbm.at[pl.ds(eb, CHUNK)], di_v)

        def mask_body(i, _):
            di = di_v[pl.ds(i * 16, 16)]
            si = si_v[pl.ds(i * 16, 16)]
            loc = di - base
            ok = (loc >= 0) & (loc < HALF)
            dil_v[pl.ds(i * 16, 16)] = jnp.where(ok, loc, 0)
            sim_v[pl.ds(i * 16, 16)] = jnp.where(ok, si, 0)
            mf_v[pl.ds(i * 16, 16)] = jnp.where(ok, 1.0, 0.0)
            return 0
        lax.fori_loop(0, CHUNK // 16, mask_body, 0, unroll=5)

        pltpu.sync_copy(g_hbm.at[sim_v], g_v)
        pltpu.sync_copy(ad_hbm.at[di_v], ad_v)

        def group_body(i, _):
            m16 = mf_v[pl.ds(i * 16, 16)]
            for j in range(16):
                c = i * 16 + j
                a_s = g_v[c, pl.ds(HID, 16)]
                a_d = ad_v[c, :]
                s = a_s + a_d
                w = jnp.exp(jnp.maximum(s, 0.2 * s)) * m16[j]
                msg_v[c, pl.ds(64, 16)] = w
                for hd in range(HEADS):
                    seg = g_v[c, pl.ds(hd * 16, 16)]
                    msg_v[c, pl.ds(hd * 16, 16)] = seg * w[12 + hd]
            return 0
        lax.fori_loop(0, CHUNK // 16, group_body, 0)

        pltpu.sync_copy(msg_v, acc_sp.at[dil_v], add=True)
        return 0
    lax.fori_loop(0, N_CHUNKS, chunk_body, 0)

    plsc.subcore_barrier()

    rbase = sid * ROWS_PER_TILE
    pltpu.sync_copy(acc_sp.at[pl.ds(rbase, ROWS_PER_TILE), :], stage_v)
    pltpu.sync_copy(stage_v, acc_hbm.at[cid, pl.ds(rbase, ROWS_PER_TILE), :])


def _sc_edge(g, ad, src, dst):
    mesh = plsc.VectorSubcoreMesh(core_axis_name="c", subcore_axis_name="s")
    f = pl.kernel(
        _sc_edge_body,
        out_type=jax.ShapeDtypeStruct((NC, HALF, ACCW), jnp.float32),
        mesh=mesh,
        compiler_params=pltpu.CompilerParams(use_tc_tiling_on_sc=False),
        scratch_types=[
            pltpu.VMEM((CHUNK, GW), jnp.float32),
            pltpu.VMEM((CHUNK, AW), jnp.float32),
            pltpu.VMEM((CHUNK, ACCW), jnp.float32),
            pltpu.VMEM((CHUNK,), jnp.int32),
            pltpu.VMEM((CHUNK,), jnp.int32),
            pltpu.VMEM((CHUNK,), jnp.int32),
            pltpu.VMEM((CHUNK,), jnp.int32),
            pltpu.VMEM((CHUNK,), jnp.float32),
            pltpu.VMEM((ROWS_PER_TILE, ACCW), jnp.float32),
            pltpu.VMEM_SHARED((HALF, ACCW), jnp.float32),
        ],
    )
    return f(g, ad, src, dst).reshape(N_PAD, ACCW)


import numpy as np

_BD_MASK = np.zeros((HID, HEADS), np.float32)
_BD_MASK[np.arange(HID), np.arange(HID) // CH] = 1.0
_E16 = np.zeros((HEADS, HID), np.float32)
_E16[np.arange(HID) // CH, np.arange(HID)] = 1.0


def _blockdiag(att):
    return jnp.asarray(_BD_MASK) * att.reshape(HID)[:, None]


def kernel(x, edge_index, W1, att_src1, att_dst1, b1, W2, att_src2, att_dst2,
           b2, Wl, bl):
    x_pad = jnp.pad(x, ((0, N_PAD - N_NODES), (0, 0)))
    src = edge_index[0]
    dst = edge_index[1]

    eye = jnp.eye(HID, dtype=jnp.float32)
    zpad = jnp.zeros((HID, AW - HEADS), jnp.float32)
    p1 = jnp.concatenate([eye, zpad, _blockdiag(att_src1)], axis=1)
    pd1 = jnp.concatenate([zpad, _blockdiag(att_dst1)], axis=1)
    p2 = jnp.concatenate([eye, zpad, _blockdiag(att_src2)], axis=1)
    pd2 = jnp.concatenate([zpad, _blockdiag(att_dst2)], axis=1)
    e16 = jnp.asarray(_E16)

    wl_pad = jnp.pad(Wl, ((0, 0), (0, 128 - N_CLASSES)))
    bl_pad = jnp.pad(bl, (0, 128 - N_CLASSES)).reshape(1, 128)

    g1, ad1 = _tc_prep(x_pad, W1, p1, pd1, D_IN)
    acc1 = _sc_edge(g1, ad1, src, dst)
    g2, ad2 = _tc_mid(acc1, e16, b1.reshape(1, HID), W2, p2, pd2)
    acc2 = _sc_edge(g2, ad2, src, dst)
    logits = _tc_final(acc2, e16, b2.reshape(1, HID), wl_pad, bl_pad)
    return logits[:N_NODES, :N_CLASSES]


# speedup vs baseline: 42.4986x; 12.3046x over previous
import jax
import jax.numpy as jnp
from jax import lax
from jax.experimental import pallas as pl
from jax.experimental.pallas import tpu as pltpu
from jax.experimental.pallas import tpu_sc as plsc

N_NODES = 10000
N_EDGES = 320000
D_IN = 128
HEADS = 4
CH = 16
HID = HEADS * CH
N_CLASSES = 2

N_PAD = 10112
GW = 80
AW = 16
ACCW = 80

NC = 2
NS = 16
HALF = N_PAD // NC
ROWS_PER_TILE = HALF // NS
E_PER_SUB = N_EDGES // NS
CHUNK = 400
N_CHUNKS = E_PER_SUB // CHUNK

BLK = N_PAD // 16
N_BLKS = 16


def _tc_prep_kernel(x_ref, w_ref, p_ref, pd_ref, g_ref, ad_ref):
    h = jnp.dot(x_ref[...], w_ref[...], preferred_element_type=jnp.float32)
    g_ref[...] = jnp.dot(h, p_ref[...], preferred_element_type=jnp.float32)
    ad_ref[...] = jnp.dot(h, pd_ref[...], preferred_element_type=jnp.float32)


def _tc_prep(x_pad, w, p, pd, d_in):
    return pl.pallas_call(
        _tc_prep_kernel,
        grid=(N_BLKS,),
        in_specs=[
            pl.BlockSpec((BLK, d_in), lambda i: (i, 0)),
            pl.BlockSpec((d_in, HID), lambda i: (0, 0)),
            pl.BlockSpec((HID, GW), lambda i: (0, 0)),
            pl.BlockSpec((HID, AW), lambda i: (0, 0)),
        ],
        out_specs=[
            pl.BlockSpec((BLK, GW), lambda i: (i, 0)),
            pl.BlockSpec((BLK, AW), lambda i: (i, 0)),
        ],
        out_shape=[
            jax.ShapeDtypeStruct((N_PAD, GW), jnp.float32),
            jax.ShapeDtypeStruct((N_PAD, AW), jnp.float32),
        ],
    )(x_pad, w, p, pd)


def _tc_mid_kernel(acc_ref, e16_ref, b_ref, w_ref, p_ref, pd_ref,
                   g_ref, ad_ref):
    a = acc_ref[...]
    o = a[:, :HID]
    d = a[:, 76:80]
    r = 1.0 / (d + 1e-16)
    rex = jnp.dot(r, e16_ref[...], preferred_element_type=jnp.float32)
    h1 = jnp.maximum(o * rex + b_ref[...], 0.0)
    t = jnp.dot(h1, w_ref[...], preferred_element_type=jnp.float32)
    g_ref[...] = jnp.dot(t, p_ref[...], preferred_element_type=jnp.float32)
    ad_ref[...] = jnp.dot(t, pd_ref[...], preferred_element_type=jnp.float32)


def _tc_mid(acc, e16, b, w, p, pd):
    return pl.pallas_call(
        _tc_mid_kernel,
        grid=(N_BLKS,),
        in_specs=[
            pl.BlockSpec((BLK, ACCW), lambda i: (i, 0)),
            pl.BlockSpec((HEADS, HID), lambda i: (0, 0)),
            pl.BlockSpec((1, HID), lambda i: (0, 0)),
            pl.BlockSpec((HID, HID), lambda i: (0, 0)),
            pl.BlockSpec((HID, GW), lambda i: (0, 0)),
            pl.BlockSpec((HID, AW), lambda i: (0, 0)),
        ],
        out_specs=[
            pl.BlockSpec((BLK, GW), lambda i: (i, 0)),
            pl.BlockSpec((BLK, AW), lambda i: (i, 0)),
        ],
        out_shape=[
            jax.ShapeDtypeStruct((N_PAD, GW), jnp.float32),
            jax.ShapeDtypeStruct((N_PAD, AW), jnp.float32),
        ],
    )(acc, e16, b, w, p, pd)


def _tc_final_kernel(acc_ref, e16_ref, b_ref, wl_ref, bl_ref, out_ref):
    a = acc_ref[...]
    o = a[:, :HID]
    d = a[:, 76:80]
    r = 1.0 / (d + 1e-16)
    rex = jnp.dot(r, e16_ref[...], preferred_element_type=jnp.float32)
    z = o * rex + b_ref[...]
    h2 = jnp.maximum(z, 0.01 * z)
    out_ref[...] = (
        jnp.dot(h2, wl_ref[...], preferred_element_type=jnp.float32)
        + bl_ref[...]
    )


def _tc_final(acc, e16, b, wl_pad, bl_pad):
    return pl.pallas_call(
        _tc_final_kernel,
        grid=(N_BLKS,),
        in_specs=[
            pl.BlockSpec((BLK, ACCW), lambda i: (i, 0)),
            pl.BlockSpec((HEADS, HID), lambda i: (0, 0)),
            pl.BlockSpec((1, HID), lambda i: (0, 0)),
            pl.BlockSpec((HID, 128), lambda i: (0, 0)),
            pl.BlockSpec((1, 128), lambda i: (0, 0)),
        ],
        out_specs=pl.BlockSpec((BLK, 128), lambda i: (i, 0)),
        out_shape=jax.ShapeDtypeStruct((N_PAD, 128), jnp.float32),
    )(acc, e16, b, wl_pad, bl_pad)


def _sc_edge_body(g_hbm, ad_hbm, src_hbm, dst_hbm, acc_hbm,
                  g_v, ad_v, msg_v, si_v, di_v, sim_v, dil_v, mf_v, acc_sp,
                  sg, sa):
    cid = lax.axis_index("c")
    sid = lax.axis_index("s")

    zero16 = jnp.zeros((16,), jnp.float32)

    def zrow(rr, _):
        for k in range(ACCW // 16):
            msg_v[rr, pl.ds(k * 16, 16)] = zero16
            g_v[rr, pl.ds(k * 16, 16)] = zero16
        return 0
    lax.fori_loop(0, CHUNK, zrow, 0, unroll=4)

    pltpu.sync_copy(msg_v.at[pl.ds(0, ROWS_PER_TILE), :],
                    acc_sp.at[pl.ds(sid * ROWS_PER_TILE, ROWS_PER_TILE), :])
    plsc.subcore_barrier()

    wbase = sid * E_PER_SUB
    base = cid * HALF

    def chunk_body(ch, _):
        eb = wbase + ch * CHUNK
        pltpu.sync_copy(src_hbm.at[pl.ds(eb, CHUNK)], si_v)
        pltpu.sync_copy(dst_hbm.at[pl.ds(eb, CHUNK)], di_v)

        def mask_body(i, _):
            sl = pl.ds(i * 16, 16)
            di = di_v[sl]
            si = si_v[sl]
            loc = di - base
            ok = (loc >= 0) & (loc < HALF)
            neg1 = jnp.full((16,), -1, jnp.int32)
            dil_v[sl] = jnp.where(ok, loc, neg1)
            sim_v[sl] = jnp.where(ok, si, neg1)
            mf_v[sl] = jnp.where(ok, 1.0, 0.0)
            return 0
        lax.fori_loop(0, CHUNK // 16, mask_body, 0, unroll=5)

        cg = pltpu.async_copy(
            g_hbm.at[plsc.Indices(sim_v, ignored_value=-1)], g_v, sg)
        ca = pltpu.async_copy(ad_hbm.at[di_v], ad_v, sa)
        cg.wait()
        ca.wait()

        def group_body(i, _):
            m16 = mf_v[pl.ds(i * 16, 16)]
            for j in range(16):
                c = i * 16 + j
                a_s = g_v[c, pl.ds(HID, 16)]
                a_d = ad_v[c, :]
                s = a_s + a_d
                w = jnp.exp(jnp.maximum(s, 0.2 * s)) * m16[j]
                msg_v[c, pl.ds(64, 16)] = w
                for hd in range(HEADS):
                    seg = g_v[c, pl.ds(hd * 16, 16)]
                    msg_v[c, pl.ds(hd * 16, 16)] = seg * w[12 + hd]
            return 0
        lax.fori_loop(0, CHUNK // 16, group_body, 0)

        pltpu.sync_copy(msg_v,
                        acc_sp.at[plsc.Indices(dil_v, ignored_value=-1)],
                        add=True)
        return 0
    lax.fori_loop(0, N_CHUNKS, chunk_body, 0)

    plsc.subcore_barrier()

    rbase = sid * ROWS_PER_TILE
    pltpu.sync_copy(acc_sp.at[pl.ds(rbase, ROWS_PER_TILE), :],
                    msg_v.at[pl.ds(0, ROWS_PER_TILE), :])
    pltpu.sync_copy(msg_v.at[pl.ds(0, ROWS_PER_TILE), :],
                    acc_hbm.at[cid, pl.ds(rbase, ROWS_PER_TILE), :])


def _sc_edge(g, ad, src, dst):
    mesh = plsc.VectorSubcoreMesh(core_axis_name="c", subcore_axis_name="s")
    f = pl.kernel(
        _sc_edge_body,
        out_type=jax.ShapeDtypeStruct((NC, HALF, ACCW), jnp.float32),
        mesh=mesh,
        compiler_params=pltpu.CompilerParams(use_tc_tiling_on_sc=False),
        scratch_types=[
            pltpu.VMEM((CHUNK, GW), jnp.float32),
            pltpu.VMEM((CHUNK, AW), jnp.float32),
            pltpu.VMEM((CHUNK, ACCW), jnp.float32),
            pltpu.VMEM((CHUNK,), jnp.int32),
            pltpu.VMEM((CHUNK,), jnp.int32),
            pltpu.VMEM((CHUNK,), jnp.int32),
            pltpu.VMEM((CHUNK,), jnp.int32),
            pltpu.VMEM((CHUNK,), jnp.float32),
            pltpu.VMEM_SHARED((HALF, ACCW), jnp.float32),
            pltpu.SemaphoreType.DMA,
            pltpu.SemaphoreType.DMA,
        ],
    )
    return f(g, ad, src, dst).reshape(N_PAD, ACCW)


import numpy as np

_BD_MASK = np.zeros((HID, HEADS), np.float32)
_BD_MASK[np.arange(HID), np.arange(HID) // CH] = 1.0
_E16 = np.zeros((HEADS, HID), np.float32)
_E16[np.arange(HID) // CH, np.arange(HID)] = 1.0


def _blockdiag(att):
    return jnp.asarray(_BD_MASK) * att.reshape(HID)[:, None]


def kernel(x, edge_index, W1, att_src1, att_dst1, b1, W2, att_src2, att_dst2,
           b2, Wl, bl):
    x_pad = jnp.pad(x, ((0, N_PAD - N_NODES), (0, 0)))
    src = edge_index[0]
    dst = edge_index[1]

    eye = jnp.eye(HID, dtype=jnp.float32)
    zpad = jnp.zeros((HID, AW - HEADS), jnp.float32)
    p1 = jnp.concatenate([eye, zpad, _blockdiag(att_src1)], axis=1)
    pd1 = jnp.concatenate([zpad, _blockdiag(att_dst1)], axis=1)
    p2 = jnp.concatenate([eye, zpad, _blockdiag(att_src2)], axis=1)
    pd2 = jnp.concatenate([zpad, _blockdiag(att_dst2)], axis=1)
    e16 = jnp.asarray(_E16)

    wl_pad = jnp.pad(Wl, ((0, 0), (0, 128 - N_CLASSES)))
    bl_pad = jnp.pad(bl, (0, 128 - N_CLASSES)).reshape(1, 128)

    g1, ad1 = _tc_prep(x_pad, W1, p1, pd1, D_IN)
    acc1 = _sc_edge(g1, ad1, src, dst)
    g2, ad2 = _tc_mid(acc1, e16, b1.reshape(1, HID), W2, p2, pd2)
    acc2 = _sc_edge(g2, ad2, src, dst)
    logits = _tc_final(acc2, e16, b2.reshape(1, HID), wl_pad, bl_pad)
    return logits[:N_NODES, :N_CLASSES]
